# Initial kernel scaffold; baseline (speedup 1.0000x reference)
import jax
import jax.numpy as jnp
from jax.experimental import pallas as pl


def kernel(adj_indices, adj_values, feats):
    raise NotImplementedError("write your pallas kernel here")


# speedup vs baseline: 3.0545x; 3.0545x over previous
import functools

import jax
import jax.numpy as jnp
from jax import lax
from jax.experimental import pallas as pl
from jax.experimental.pallas import tpu as pltpu
from jax.experimental.pallas import tpu_sc as plsc

N_CORES = 2
N_TILES = 16
LANES = 16
CHUNK = 80


def _sc_partials(cols, rows, vals_rep, feats):
    n_nodes, d = feats.shape
    n_edges = cols.shape[0]
    epc = n_edges // N_CORES
    ept = epc // N_TILES
    nch = ept // CHUNK
    npt = ((n_nodes + N_TILES * 32 - 1) // (N_TILES * 32)) * 32
    n_pad = npt * N_TILES

    mesh = plsc.VectorSubcoreMesh(core_axis_name="c", subcore_axis_name="s")

    @functools.partial(
        pl.kernel,
        out_type=jax.ShapeDtypeStruct((N_CORES, n_pad, d), jnp.float32),
        mesh=mesh,
        scratch_types=[
            pltpu.VMEM((CHUNK,), jnp.int32),
            pltpu.VMEM((CHUNK,), jnp.int32),
            pltpu.VMEM((CHUNK, LANES), jnp.float32),
            pltpu.VMEM((CHUNK, d), jnp.float32),
            pltpu.VMEM((npt // 4, d), jnp.float32),
            pltpu.VMEM_SHARED((n_pad, d), jnp.float32),
            pltpu.SemaphoreType.DMA,
        ],
    )
    def k(cols_hbm, rows_hbm, vals_hbm, feats_hbm, out_hbm,
          col_v, row_v, val_v, gath_v, stage_v, acc_sh, sem):
        cid = lax.axis_index("c")
        sid = lax.axis_index("s")

        zero = jnp.zeros((LANES,), jnp.float32)
        stg = npt // 4

        def zrow(r, carry):
            for q in range(d // LANES):
                stage_v[r, pl.ds(q * LANES, LANES)] = zero
            return carry

        lax.fori_loop(0, stg, zrow, 0)
        for w in range(4):
            pltpu.sync_copy(stage_v, acc_sh.at[pl.ds(sid * npt + w * stg, stg)])
        plsc.subcore_barrier()

        base = cid * epc + sid * ept

        def chunk_body(j, carry):
            off = base + j * CHUNK
            pltpu.sync_copy(cols_hbm.at[pl.ds(off, CHUNK)], col_v)
            pltpu.sync_copy(rows_hbm.at[pl.ds(off, CHUNK)], row_v)
            pltpu.sync_copy(vals_hbm.at[pl.ds(off, CHUNK)], val_v)
            pltpu.async_copy(feats_hbm.at[col_v], gath_v, sem).wait()

            def scale(e, c2):
                v = val_v[e]
                for q in range(d // LANES):
                    gath_v[e, pl.ds(q * LANES, LANES)] = (
                        gath_v[e, pl.ds(q * LANES, LANES)] * v)
                return c2

            lax.fori_loop(0, CHUNK, scale, 0)
            pltpu.sync_copy(gath_v, acc_sh.at[row_v], add=True)
            return carry

        lax.fori_loop(0, nch, chunk_body, 0)
        plsc.subcore_barrier()

        for w in range(4):
            pltpu.sync_copy(acc_sh.at[pl.ds(sid * npt + w * stg, stg)], stage_v)
            pltpu.sync_copy(stage_v, out_hbm.at[cid, pl.ds(sid * npt + w * stg, stg)])

    return k(cols, rows, vals_rep, feats)


def _tc_sum(p0, p1):
    n, d = p0.shape
    blk = 1000

    def body(a_ref, b_ref, o_ref):
        o_ref[...] = a_ref[...] + b_ref[...]

    spec = pl.BlockSpec((blk, d), lambda i: (i, 0))
    return pl.pallas_call(
        body,
        grid=(n // blk,),
        in_specs=[spec, spec],
        out_specs=spec,
        out_shape=jax.ShapeDtypeStruct((n, d), jnp.float32),
    )(p0, p1)


def kernel(adj_indices, adj_values, feats):
    n_nodes, d_feat = feats.shape
    n_edges = adj_values.shape[0]

    rows = adj_indices[0].astype(jnp.int32)
    cols = adj_indices[1].astype(jnp.int32)
    vals_rep = jnp.broadcast_to(adj_values[:, None], (n_edges, LANES))

    parts = _sc_partials(cols, rows, vals_rep, feats)
    return _tc_sum(parts[0, :n_nodes], parts[1, :n_nodes])


# speedup vs baseline: 4.7220x; 1.5459x over previous
import functools

import jax
import jax.numpy as jnp
from jax import lax
from jax.experimental import pallas as pl
from jax.experimental.pallas import tpu as pltpu
from jax.experimental.pallas import tpu_sc as plsc

N_CORES = 2
N_TILES = 16
LANES = 16
CHUNK = 80


def _sc_partials(cols, rows, vals_rep, feats):
    n_nodes, d = feats.shape
    n_edges = cols.shape[0]
    epc = n_edges // N_CORES
    ept = epc // N_TILES
    nch = ept // CHUNK
    npt = ((n_nodes + N_TILES * 32 - 1) // (N_TILES * 32)) * 32
    n_pad = npt * N_TILES

    mesh = plsc.VectorSubcoreMesh(core_axis_name="c", subcore_axis_name="s")

    idx_t = pltpu.VMEM((CHUNK,), jnp.int32)
    val_t = pltpu.VMEM((CHUNK, LANES), jnp.float32)
    gath_t = pltpu.VMEM((CHUNK, d), jnp.float32)

    @functools.partial(
        pl.kernel,
        out_type=jax.ShapeDtypeStruct((N_CORES, n_pad, d), jnp.float32),
        mesh=mesh,
        scratch_types=[
            idx_t, idx_t,
            idx_t, idx_t,
            val_t, val_t,
            gath_t, gath_t,
            pltpu.VMEM_SHARED((n_pad, d), jnp.float32),
            pltpu.SemaphoreType.DMA, pltpu.SemaphoreType.DMA,
            pltpu.SemaphoreType.DMA, pltpu.SemaphoreType.DMA,
        ],
    )
    def k(cols_hbm, rows_hbm, vals_hbm, feats_hbm, out_hbm,
          col0, col1, row0, row1, val0, val1, g0, g1,
          acc_sh, si0, si1, sg0, sg1):
        cid = lax.axis_index("c")
        sid = lax.axis_index("s")
        col_v = (col0, col1)
        row_v = (row0, row1)
        val_v = (val0, val1)
        gath_v = (g0, g1)
        sem_i = (si0, si1)
        sem_g = (sg0, sg1)

        zero = jnp.zeros((LANES,), jnp.float32)
        n_w = npt // CHUNK

        def zrow(r, carry):
            for q in range(d // LANES):
                g0[r, pl.ds(q * LANES, LANES)] = zero
            return carry

        lax.fori_loop(0, CHUNK, zrow, 0)
        for w in range(n_w):
            pltpu.sync_copy(g0, acc_sh.at[pl.ds(sid * npt + w * CHUNK, CHUNK)])
        plsc.subcore_barrier()

        base = cid * epc + sid * ept

        def issue_idx(c, b):
            off = base + c * CHUNK
            pltpu.async_copy(cols_hbm.at[pl.ds(off, CHUNK)], col_v[b], sem_i[b])
            pltpu.async_copy(rows_hbm.at[pl.ds(off, CHUNK)], row_v[b], sem_i[b])
            pltpu.async_copy(vals_hbm.at[pl.ds(off, CHUNK)], val_v[b], sem_i[b])

        def wait_idx(b):
            d0 = pl.ds(0, CHUNK)
            pltpu.make_async_copy(cols_hbm.at[d0], col_v[b], sem_i[b]).wait()
            pltpu.make_async_copy(rows_hbm.at[d0], row_v[b], sem_i[b]).wait()
            pltpu.make_async_copy(vals_hbm.at[d0], val_v[b], sem_i[b]).wait()

        def scale_scatter(b):
            def scale(e, c2):
                v = val_v[b][e]
                for q in range(d // LANES):
                    gath_v[b][e, pl.ds(q * LANES, LANES)] = (
                        gath_v[b][e, pl.ds(q * LANES, LANES)] * v)
                return c2

            lax.fori_loop(0, CHUNK, scale, 0)
            pltpu.sync_copy(gath_v[b], acc_sh.at[row_v[b]], add=True)

        def issue_gather(c, b):
            del c
            pltpu.async_copy(feats_hbm.at[col_v[b]], gath_v[b], sem_g[b])

        def wait_gather(b):
            pltpu.make_async_copy(feats_hbm.at[col_v[b]], gath_v[b],
                                  sem_g[b]).wait()

        issue_idx(0, 0)

        def pipe(c2, carry):
            for b in range(2):
                c = 2 * c2 + b

                @pl.when(c < nch)
                def _(c=c, b=b):
                    wait_idx(b)
                    issue_gather(c, b)

                @pl.when(jnp.logical_and(c > 0, c <= nch))
                def _(c=c, b=b):
                    wait_gather(1 - b)
                    scale_scatter(1 - b)

                @pl.when(c + 1 < nch)
                def _(c=c, b=b):
                    issue_idx(c + 1, 1 - b)

            return carry

        lax.fori_loop(0, (nch + 2) // 2, pipe, 0)

        plsc.subcore_barrier()

        for w in range(n_w):
            off = pl.ds(sid * npt + w * CHUNK, CHUNK)
            pltpu.sync_copy(acc_sh.at[off], g0)
            pltpu.sync_copy(g0, out_hbm.at[cid, off])

    return k(cols, rows, vals_rep, feats)


def _tc_sum(p0, p1):
    n, d = p0.shape
    blk = 1000

    def body(a_ref, b_ref, o_ref):
        o_ref[...] = a_ref[...] + b_ref[...]

    spec = pl.BlockSpec((blk, d), lambda i: (i, 0))
    return pl.pallas_call(
        body,
        grid=(n // blk,),
        in_specs=[spec, spec],
        out_specs=spec,
        out_shape=jax.ShapeDtypeStruct((n, d), jnp.float32),
    )(p0, p1)


def kernel(adj_indices, adj_values, feats):
    n_nodes, d_feat = feats.shape
    n_edges = adj_values.shape[0]

    rows = adj_indices[0].astype(jnp.int32)
    cols = adj_indices[1].astype(jnp.int32)
    vals_rep = jnp.broadcast_to(adj_values[:, None], (n_edges, LANES))

    parts = _sc_partials(cols, rows, vals_rep, feats)
    return _tc_sum(parts[0, :n_nodes], parts[1, :n_nodes])
